# baseline (speedup 1.0000x reference)
import functools

import jax
import jax.numpy as jnp
from jax import lax
from jax.experimental import pallas as pl
from jax.experimental.pallas import tpu as pltpu
from jax.experimental.pallas import tpu_sc as plsc

N = 10000
D = 128
E = 320000

NC = 2
NS = 16
NW = NC * NS
CHUNK = 64
CHH = 80
CH = 2 * CHH
E_PAD = NW * CH * CHUNK
N_PAD = 10112
RPT = N_PAD // NS
DEGW = 128


def _sc_agg_body(z_hbm, src_hbm, dst_hbm, zrows_hbm, agg_out,
                 src_v, dst_v, rows_v, agg_sh, sem):
    cid = lax.axis_index("c")
    sid = lax.axis_index("s")
    wid = sid * NC + cid
    base = sid * RPT

    pltpu.sync_copy(zrows_hbm, agg_sh.at[pl.ds(base, RPT)])
    plsc.subcore_barrier()

    for h in (0, 1):
        pltpu.sync_copy(src_hbm.at[wid, pl.ds(h * CHH, CHH)], src_v)
        pltpu.sync_copy(dst_hbm.at[wid, pl.ds(h * CHH, CHH)], dst_v)

        def chunk_body(j, carry):
            @pl.when(j < CHH)
            def _():
                off = (j % 2) * CHUNK
                pltpu.async_copy(z_hbm.at[src_v.at[j]],
                                 rows_v.at[pl.ds(off, CHUNK)], sem.at[j % 2])

            @pl.when(j > 0)
            def _():
                jp = j - 1
                off = (jp % 2) * CHUNK
                buf = rows_v.at[pl.ds(off, CHUNK)]
                pltpu.make_async_copy(z_hbm.at[src_v.at[jp]], buf,
                                      sem.at[jp % 2]).wait()
                pltpu.sync_copy(buf, agg_sh.at[dst_v.at[jp]], add=True)

            return carry

        lax.fori_loop(0, CHH + 1, chunk_body, 0)

    plsc.subcore_barrier()
    pltpu.sync_copy(agg_sh.at[pl.ds(base, RPT)], agg_out.at[cid, pl.ds(base, RPT)])


def _sc_deg_body(dst_hbm, zdeg_hbm, ones_hbm, deg_out,
                 dst_v, ones_v, deg_sh):
    cid = lax.axis_index("c")
    sid = lax.axis_index("s")
    wid = sid * NC + cid
    base = sid * RPT

    pltpu.sync_copy(zdeg_hbm, deg_sh.at[pl.ds(base, RPT)])
    pltpu.sync_copy(ones_hbm, ones_v)
    pltpu.sync_copy(dst_hbm.at[wid], dst_v)
    plsc.subcore_barrier()

    def chunk_body(j, carry):
        pltpu.sync_copy(ones_v, deg_sh.at[dst_v.at[j]], add=True)
        return carry

    lax.fori_loop(0, CH, chunk_body, 0)
    plsc.subcore_barrier()
    pltpu.sync_copy(deg_sh.at[pl.ds(base, RPT)], deg_out.at[cid, pl.ds(base, RPT)])


@functools.lru_cache(maxsize=None)
def _sc_kernels():
    mesh = plsc.VectorSubcoreMesh(
        core_axis_name="c", subcore_axis_name="s", num_cores=NC, num_subcores=NS)
    sc_deg = pl.kernel(
        _sc_deg_body,
        out_type=jax.ShapeDtypeStruct((NC, N_PAD, DEGW), jnp.float32),
        mesh=mesh,
        scratch_types=[
            pltpu.VMEM((CH, CHUNK), jnp.int32),
            pltpu.VMEM((CHUNK, DEGW), jnp.float32),
            pltpu.VMEM_SHARED((N_PAD, DEGW), jnp.float32),
        ],
    )
    sc_agg = pl.kernel(
        _sc_agg_body,
        out_type=jax.ShapeDtypeStruct((NC, N_PAD, D), jnp.float32),
        mesh=mesh,
        scratch_types=[
            pltpu.VMEM((CHH, CHUNK), jnp.int32),
            pltpu.VMEM((CHH, CHUNK), jnp.int32),
            pltpu.VMEM((2 * CHUNK, D), jnp.float32),
            pltpu.VMEM_SHARED((N_PAD, D), jnp.float32),
            pltpu.SemaphoreType.DMA((2,)),
        ],
    )
    return sc_deg, sc_agg


R = 2000


def _tc_in_body(x_ref, wl_ref, wr_ref, z_ref, s_ref):
    h = x_ref[...]
    z_ref[...] = jnp.dot(h, wl_ref[...], preferred_element_type=jnp.float32)
    s_ref[...] = jnp.dot(h, wr_ref[...], preferred_element_type=jnp.float32)


def _tc_in(x, wl, wr):
    return pl.pallas_call(
        _tc_in_body,
        grid=(N // R,),
        in_specs=[
            pl.BlockSpec((R, D), lambda i: (i, 0)),
            pl.BlockSpec((D, D), lambda i: (0, 0)),
            pl.BlockSpec((D, D), lambda i: (0, 0)),
        ],
        out_specs=[
            pl.BlockSpec((R, D), lambda i: (i, 0)),
            pl.BlockSpec((R, D), lambda i: (i, 0)),
        ],
        out_shape=[
            jax.ShapeDtypeStruct((N, D), jnp.float32),
            jax.ShapeDtypeStruct((N, D), jnp.float32),
        ],
    )(x, wl, wr)


def _combine_mean(agg_ref, deg_ref, b_ref, s_ref):
    a = agg_ref[...]
    dg = deg_ref[...]
    aggsum = a[0] + a[1]
    deg = dg[0, :, 0:1] + dg[1, :, 0:1]
    mean = aggsum / jnp.maximum(deg, 1.0)
    return mean + b_ref[...] + s_ref[...]


def _tc_mid_body(agg_ref, deg_ref, b_ref, s_ref, wl_ref, wr_ref, z_ref, s2_ref):
    h = jnp.maximum(_combine_mean(agg_ref, deg_ref, b_ref, s_ref), 0.0)
    z_ref[...] = jnp.dot(h, wl_ref[...], preferred_element_type=jnp.float32)
    s2_ref[...] = jnp.dot(h, wr_ref[...], preferred_element_type=jnp.float32)


def _tc_mid(agg, deg, b, s, wl, wr):
    return pl.pallas_call(
        _tc_mid_body,
        grid=(N // R,),
        in_specs=[
            pl.BlockSpec((NC, R, D), lambda i: (0, i, 0)),
            pl.BlockSpec((NC, R, DEGW), lambda i: (0, i, 0)),
            pl.BlockSpec((1, D), lambda i: (0, 0)),
            pl.BlockSpec((R, D), lambda i: (i, 0)),
            pl.BlockSpec((D, D), lambda i: (0, 0)),
            pl.BlockSpec((D, D), lambda i: (0, 0)),
        ],
        out_specs=[
            pl.BlockSpec((R, D), lambda i: (i, 0)),
            pl.BlockSpec((R, D), lambda i: (i, 0)),
        ],
        out_shape=[
            jax.ShapeDtypeStruct((N, D), jnp.float32),
            jax.ShapeDtypeStruct((N, D), jnp.float32),
        ],
    )(agg, deg, b, s, wl, wr)


def _tc_final_body(agg_ref, deg_ref, b_ref, s_ref, out_ref):
    pre = _combine_mean(agg_ref, deg_ref, b_ref, s_ref)
    m = jnp.max(pre, axis=-1, keepdims=True)
    e = jnp.exp(pre - m)
    out_ref[...] = e / jnp.sum(e, axis=-1, keepdims=True)


def _tc_final(agg, deg, b, s):
    return pl.pallas_call(
        _tc_final_body,
        grid=(N // R,),
        in_specs=[
            pl.BlockSpec((NC, R, D), lambda i: (0, i, 0)),
            pl.BlockSpec((NC, R, DEGW), lambda i: (0, i, 0)),
            pl.BlockSpec((1, D), lambda i: (0, 0)),
            pl.BlockSpec((R, D), lambda i: (i, 0)),
        ],
        out_specs=pl.BlockSpec((R, D), lambda i: (i, 0)),
        out_shape=jax.ShapeDtypeStruct((N, D), jnp.float32),
    )(agg, deg, b, s)


def kernel(x, edge_index, W1l, b1, W1r, W2l, b2, W2r, W3l, b3, W3r):
    pad = E_PAD - E
    src = jnp.concatenate([edge_index[0], jnp.zeros((pad,), jnp.int32)])
    dst = jnp.concatenate([edge_index[1], jnp.full((pad,), N, jnp.int32)])
    src_r = src.reshape(NW, CH, CHUNK)
    dst_r = dst.reshape(NW, CH, CHUNK)

    zrows = jnp.zeros((RPT, D), jnp.float32)
    zdeg = jnp.zeros((RPT, DEGW), jnp.float32)
    ones = jnp.ones((CHUNK, DEGW), jnp.float32)

    b1r = b1.reshape(1, D)
    b2r = b2.reshape(1, D)
    b3r = b3.reshape(1, D)

    sc_deg, sc_agg = _sc_kernels()
    dega = sc_deg(dst_r, zdeg, ones)
    z1, s1 = _tc_in(x, W1l, W1r)
    agg1 = sc_agg(z1, src_r, dst_r, zrows)
    z2, s2 = _tc_mid(agg1, dega, b1r, s1, W2l, W2r)
    agg2 = sc_agg(z2, src_r, dst_r, zrows)
    z3, s3 = _tc_mid(agg2, dega, b2r, s2, W3l, W3r)
    agg3 = sc_agg(z3, src_r, dst_r, zrows)
    return _tc_final(agg3, dega, b3r, s3)


# speedup vs baseline: 2.6592x; 2.6592x over previous
import functools

import jax
import jax.numpy as jnp
from jax import lax
from jax.experimental import pallas as pl
from jax.experimental.pallas import tpu as pltpu
from jax.experimental.pallas import tpu_sc as plsc

N = 10000
D = 128
E = 320000

NC = 2
NS = 16
NW = NC * NS
CHUNK = 64
CHH = 80
CH = 2 * CHH
E_PAD = NW * CH * CHUNK
N_PAD = 10112
RPT = N_PAD // NS
DEGW = 128


def _sc_agg_body(z_hbm, src_hbm, dst_hbm, zrows_hbm, agg_out,
                 src_v, dst_v, rows_v, agg_sh, sem):
    cid = lax.axis_index("c")
    sid = lax.axis_index("s")
    wid = sid * NC + cid
    base = sid * RPT

    pltpu.sync_copy(zrows_hbm, agg_sh.at[pl.ds(base, RPT)])
    plsc.subcore_barrier()

    for h in (0, 1):
        pltpu.sync_copy(src_hbm.at[wid, pl.ds(h * CHH, CHH)], src_v)
        pltpu.sync_copy(dst_hbm.at[wid, pl.ds(h * CHH, CHH)], dst_v)

        def chunk_body(j, carry):
            @pl.when(j < CHH)
            def _():
                off = (j % 2) * CHUNK
                pltpu.async_copy(z_hbm.at[src_v.at[j]],
                                 rows_v.at[pl.ds(off, CHUNK)], sem.at[j % 2])

            @pl.when(j > 0)
            def _():
                jp = j - 1
                off = (jp % 2) * CHUNK
                buf = rows_v.at[pl.ds(off, CHUNK)]
                pltpu.make_async_copy(z_hbm.at[src_v.at[jp]], buf,
                                      sem.at[jp % 2]).wait()
                pltpu.sync_copy(buf, agg_sh.at[dst_v.at[jp]], add=True)

            return carry

        lax.fori_loop(0, CHH + 1, chunk_body, 0)

    plsc.subcore_barrier()
    pltpu.sync_copy(agg_sh.at[pl.ds(base, RPT)], agg_out.at[cid, pl.ds(base, RPT)])


def _sc_deg_body(dst_hbm, zdeg_hbm, ones_hbm, deg_out,
                 dst_v, ones_v, deg_sh):
    cid = lax.axis_index("c")
    sid = lax.axis_index("s")
    wid = sid * NC + cid
    base = sid * RPT

    pltpu.sync_copy(zdeg_hbm, deg_sh.at[pl.ds(base, RPT)])
    pltpu.sync_copy(ones_hbm, ones_v)
    pltpu.sync_copy(dst_hbm.at[wid], dst_v)
    plsc.subcore_barrier()

    def chunk_body(j, carry):
        pltpu.sync_copy(ones_v, deg_sh.at[dst_v.at[j]], add=True)
        return carry

    lax.fori_loop(0, CH, chunk_body, 0)
    plsc.subcore_barrier()
    pltpu.sync_copy(deg_sh.at[pl.ds(base, RPT)], deg_out.at[cid, pl.ds(base, RPT)])


@functools.lru_cache(maxsize=None)
def _sc_kernels():
    mesh = plsc.VectorSubcoreMesh(
        core_axis_name="c", subcore_axis_name="s", num_cores=NC, num_subcores=NS)
    sc_deg = pl.kernel(
        _sc_deg_body,
        out_type=jax.ShapeDtypeStruct((NC, N_PAD, DEGW), jnp.float32),
        mesh=mesh,
        scratch_types=[
            pltpu.VMEM((CH, CHUNK), jnp.int32),
            pltpu.VMEM((CHUNK, DEGW), jnp.float32),
            pltpu.VMEM_SHARED((N_PAD, DEGW), jnp.float32),
        ],
    )
    sc_agg = pl.kernel(
        _sc_agg_body,
        out_type=jax.ShapeDtypeStruct((NC, N_PAD, D), jnp.float32),
        mesh=mesh,
        scratch_types=[
            pltpu.VMEM((CHH, CHUNK), jnp.int32),
            pltpu.VMEM((CHH, CHUNK), jnp.int32),
            pltpu.VMEM((2 * CHUNK, D), jnp.float32),
            pltpu.VMEM_SHARED((N_PAD, D), jnp.float32),
            pltpu.SemaphoreType.DMA((2,)),
        ],
    )
    return sc_deg, sc_agg


R = 2000


def _tc_in_body(x_ref, wl_ref, wr_ref, z_ref, s_ref):
    h = x_ref[...]
    z_ref[...] = jnp.dot(h, wl_ref[...], preferred_element_type=jnp.float32)
    s_ref[...] = jnp.dot(h, wr_ref[...], preferred_element_type=jnp.float32)


def _tc_in(x, wl, wr):
    return pl.pallas_call(
        _tc_in_body,
        grid=(N // R,),
        in_specs=[
            pl.BlockSpec((R, D), lambda i: (i, 0)),
            pl.BlockSpec((D, D), lambda i: (0, 0)),
            pl.BlockSpec((D, D), lambda i: (0, 0)),
        ],
        out_specs=[
            pl.BlockSpec((R, D), lambda i: (i, 0)),
            pl.BlockSpec((R, D), lambda i: (i, 0)),
        ],
        out_shape=[
            jax.ShapeDtypeStruct((N, D), jnp.float32),
            jax.ShapeDtypeStruct((N, D), jnp.float32),
        ],
    )(x, wl, wr)


def _combine_mean(agg_ref, deg_ref, b_ref, s_ref):
    a = agg_ref[...]
    dg = deg_ref[...]
    aggsum = a[0] + a[1]
    deg = dg[0, :, 0:1] + dg[1, :, 0:1]
    mean = aggsum / jnp.maximum(deg, 1.0)
    return mean + b_ref[...] + s_ref[...]


def _tc_mid_body(agg_ref, deg_ref, b_ref, s_ref, wl_ref, wr_ref, z_ref, s2_ref):
    h = jnp.maximum(_combine_mean(agg_ref, deg_ref, b_ref, s_ref), 0.0)
    z_ref[...] = jnp.dot(h, wl_ref[...], preferred_element_type=jnp.float32)
    s2_ref[...] = jnp.dot(h, wr_ref[...], preferred_element_type=jnp.float32)


def _tc_mid(agg, deg, b, s, wl, wr):
    return pl.pallas_call(
        _tc_mid_body,
        grid=(N // R,),
        in_specs=[
            pl.BlockSpec((NC, R, D), lambda i: (0, i, 0)),
            pl.BlockSpec((NC, R, DEGW), lambda i: (0, i, 0)),
            pl.BlockSpec((1, D), lambda i: (0, 0)),
            pl.BlockSpec((R, D), lambda i: (i, 0)),
            pl.BlockSpec((D, D), lambda i: (0, 0)),
            pl.BlockSpec((D, D), lambda i: (0, 0)),
        ],
        out_specs=[
            pl.BlockSpec((R, D), lambda i: (i, 0)),
            pl.BlockSpec((R, D), lambda i: (i, 0)),
        ],
        out_shape=[
            jax.ShapeDtypeStruct((N, D), jnp.float32),
            jax.ShapeDtypeStruct((N, D), jnp.float32),
        ],
    )(agg, deg, b, s, wl, wr)


def _tc_final_body(agg_ref, deg_ref, b_ref, s_ref, out_ref):
    pre = _combine_mean(agg_ref, deg_ref, b_ref, s_ref)
    m = jnp.max(pre, axis=-1, keepdims=True)
    e = jnp.exp(pre - m)
    out_ref[...] = e / jnp.sum(e, axis=-1, keepdims=True)


def _tc_final(agg, deg, b, s):
    return pl.pallas_call(
        _tc_final_body,
        grid=(N // R,),
        in_specs=[
            pl.BlockSpec((NC, R, D), lambda i: (0, i, 0)),
            pl.BlockSpec((NC, R, DEGW), lambda i: (0, i, 0)),
            pl.BlockSpec((1, D), lambda i: (0, 0)),
            pl.BlockSpec((R, D), lambda i: (i, 0)),
        ],
        out_specs=pl.BlockSpec((R, D), lambda i: (i, 0)),
        out_shape=jax.ShapeDtypeStruct((N, D), jnp.float32),
    )(agg, deg, b, s)


def kernel(x, edge_index, W1l, b1, W1r, W2l, b2, W2r, W3l, b3, W3r):
    pad = E_PAD - E
    pad_src = (jnp.arange(pad, dtype=jnp.int32) * 37) % N
    pad_dst = N + (jnp.arange(pad, dtype=jnp.int32) % (N_PAD - N))
    src = jnp.concatenate([edge_index[0], pad_src])
    dst = jnp.concatenate([edge_index[1], pad_dst])
    src_r = src.reshape(NW, CH, CHUNK)
    dst_r = dst.reshape(NW, CH, CHUNK)

    zrows = jnp.zeros((RPT, D), jnp.float32)
    zdeg = jnp.zeros((RPT, DEGW), jnp.float32)
    ones = jnp.ones((CHUNK, DEGW), jnp.float32)

    b1r = b1.reshape(1, D)
    b2r = b2.reshape(1, D)
    b3r = b3.reshape(1, D)

    sc_deg, sc_agg = _sc_kernels()
    dega = sc_deg(dst_r, zdeg, ones)
    z1, s1 = _tc_in(x, W1l, W1r)
    agg1 = sc_agg(z1, src_r, dst_r, zrows)
    z2, s2 = _tc_mid(agg1, dega, b1r, s1, W2l, W2r)
    agg2 = sc_agg(z2, src_r, dst_r, zrows)
    z3, s3 = _tc_mid(agg2, dega, b2r, s2, W3l, W3r)
    agg3 = sc_agg(z3, src_r, dst_r, zrows)
    return _tc_final(agg3, dega, b3r, s3)


# speedup vs baseline: 3.0235x; 1.1370x over previous
import functools

import jax
import jax.numpy as jnp
from jax import lax
from jax.experimental import pallas as pl
from jax.experimental.pallas import tpu as pltpu
from jax.experimental.pallas import tpu_sc as plsc

N = 10000
D = 128
E = 320000

NC = 2
NS = 16
NW = NC * NS
CHUNK = 128
CHH = 40
CH = 2 * CHH
E_PAD = NW * CH * CHUNK
N_PAD = 10112
RPT = N_PAD // NS
DEGW = 128


def _sc_agg_body(z_hbm, src_hbm, dst_hbm, zrows_hbm, agg_out,
                 src_v, dst_v, rows_v, agg_sh, sem):
    cid = lax.axis_index("c")
    sid = lax.axis_index("s")
    wid = sid * NC + cid
    base = sid * RPT

    pltpu.sync_copy(zrows_hbm, agg_sh.at[pl.ds(base, RPT)])
    plsc.subcore_barrier()

    for h in (0, 1):
        pltpu.sync_copy(src_hbm.at[wid, pl.ds(h * CHH, CHH)], src_v)
        pltpu.sync_copy(dst_hbm.at[wid, pl.ds(h * CHH, CHH)], dst_v)

        def chunk_body(j, carry):
            @pl.when(j < CHH)
            def _():
                off = (j % 2) * CHUNK
                pltpu.async_copy(z_hbm.at[src_v.at[j]],
                                 rows_v.at[pl.ds(off, CHUNK)], sem.at[j % 2])

            @pl.when(j > 0)
            def _():
                jp = j - 1
                off = (jp % 2) * CHUNK
                buf = rows_v.at[pl.ds(off, CHUNK)]
                pltpu.make_async_copy(z_hbm.at[src_v.at[jp]], buf,
                                      sem.at[jp % 2]).wait()
                pltpu.sync_copy(buf, agg_sh.at[dst_v.at[jp]], add=True)

            return carry

        lax.fori_loop(0, CHH + 1, chunk_body, 0)

    plsc.subcore_barrier()
    pltpu.sync_copy(agg_sh.at[pl.ds(base, RPT)], agg_out.at[cid, pl.ds(base, RPT)])


def _sc_deg_body(dst_hbm, zdeg_hbm, ones_hbm, deg_out,
                 dst_v, ones_v, deg_sh):
    cid = lax.axis_index("c")
    sid = lax.axis_index("s")
    wid = sid * NC + cid
    base = sid * RPT

    pltpu.sync_copy(zdeg_hbm, deg_sh.at[pl.ds(base, RPT)])
    pltpu.sync_copy(ones_hbm, ones_v)
    pltpu.sync_copy(dst_hbm.at[wid], dst_v)
    plsc.subcore_barrier()

    def chunk_body(j, carry):
        pltpu.sync_copy(ones_v, deg_sh.at[dst_v.at[j]], add=True)
        return carry

    lax.fori_loop(0, CH, chunk_body, 0)
    plsc.subcore_barrier()
    pltpu.sync_copy(deg_sh.at[pl.ds(base, RPT)], deg_out.at[cid, pl.ds(base, RPT)])


@functools.lru_cache(maxsize=None)
def _sc_kernels():
    mesh = plsc.VectorSubcoreMesh(
        core_axis_name="c", subcore_axis_name="s", num_cores=NC, num_subcores=NS)
    sc_deg = pl.kernel(
        _sc_deg_body,
        out_type=jax.ShapeDtypeStruct((NC, N_PAD, DEGW), jnp.float32),
        mesh=mesh,
        scratch_types=[
            pltpu.VMEM((CH, CHUNK), jnp.int32),
            pltpu.VMEM((CHUNK, DEGW), jnp.float32),
            pltpu.VMEM_SHARED((N_PAD, DEGW), jnp.float32),
        ],
    )
    sc_agg = pl.kernel(
        _sc_agg_body,
        out_type=jax.ShapeDtypeStruct((NC, N_PAD, D), jnp.float32),
        mesh=mesh,
        scratch_types=[
            pltpu.VMEM((CHH, CHUNK), jnp.int32),
            pltpu.VMEM((CHH, CHUNK), jnp.int32),
            pltpu.VMEM((2 * CHUNK, D), jnp.float32),
            pltpu.VMEM_SHARED((N_PAD, D), jnp.float32),
            pltpu.SemaphoreType.DMA((2,)),
        ],
    )
    return sc_deg, sc_agg


R = 2000


def _tc_in_body(x_ref, wl_ref, wr_ref, z_ref, s_ref):
    h = x_ref[...]
    z_ref[...] = jnp.dot(h, wl_ref[...], preferred_element_type=jnp.float32)
    s_ref[...] = jnp.dot(h, wr_ref[...], preferred_element_type=jnp.float32)


def _tc_in(x, wl, wr):
    return pl.pallas_call(
        _tc_in_body,
        grid=(N // R,),
        in_specs=[
            pl.BlockSpec((R, D), lambda i: (i, 0)),
            pl.BlockSpec((D, D), lambda i: (0, 0)),
            pl.BlockSpec((D, D), lambda i: (0, 0)),
        ],
        out_specs=[
            pl.BlockSpec((R, D), lambda i: (i, 0)),
            pl.BlockSpec((R, D), lambda i: (i, 0)),
        ],
        out_shape=[
            jax.ShapeDtypeStruct((N, D), jnp.float32),
            jax.ShapeDtypeStruct((N, D), jnp.float32),
        ],
    )(x, wl, wr)


def _combine_mean(agg_ref, deg_ref, b_ref, s_ref):
    a = agg_ref[...]
    dg = deg_ref[...]
    aggsum = a[0] + a[1]
    deg = dg[0, :, 0:1] + dg[1, :, 0:1]
    mean = aggsum / jnp.maximum(deg, 1.0)
    return mean + b_ref[...] + s_ref[...]


def _tc_mid_body(agg_ref, deg_ref, b_ref, s_ref, wl_ref, wr_ref, z_ref, s2_ref):
    h = jnp.maximum(_combine_mean(agg_ref, deg_ref, b_ref, s_ref), 0.0)
    z_ref[...] = jnp.dot(h, wl_ref[...], preferred_element_type=jnp.float32)
    s2_ref[...] = jnp.dot(h, wr_ref[...], preferred_element_type=jnp.float32)


def _tc_mid(agg, deg, b, s, wl, wr):
    return pl.pallas_call(
        _tc_mid_body,
        grid=(N // R,),
        in_specs=[
            pl.BlockSpec((NC, R, D), lambda i: (0, i, 0)),
            pl.BlockSpec((NC, R, DEGW), lambda i: (0, i, 0)),
            pl.BlockSpec((1, D), lambda i: (0, 0)),
            pl.BlockSpec((R, D), lambda i: (i, 0)),
            pl.BlockSpec((D, D), lambda i: (0, 0)),
            pl.BlockSpec((D, D), lambda i: (0, 0)),
        ],
        out_specs=[
            pl.BlockSpec((R, D), lambda i: (i, 0)),
            pl.BlockSpec((R, D), lambda i: (i, 0)),
        ],
        out_shape=[
            jax.ShapeDtypeStruct((N, D), jnp.float32),
            jax.ShapeDtypeStruct((N, D), jnp.float32),
        ],
    )(agg, deg, b, s, wl, wr)


def _tc_final_body(agg_ref, deg_ref, b_ref, s_ref, out_ref):
    pre = _combine_mean(agg_ref, deg_ref, b_ref, s_ref)
    m = jnp.max(pre, axis=-1, keepdims=True)
    e = jnp.exp(pre - m)
    out_ref[...] = e / jnp.sum(e, axis=-1, keepdims=True)


def _tc_final(agg, deg, b, s):
    return pl.pallas_call(
        _tc_final_body,
        grid=(N // R,),
        in_specs=[
            pl.BlockSpec((NC, R, D), lambda i: (0, i, 0)),
            pl.BlockSpec((NC, R, DEGW), lambda i: (0, i, 0)),
            pl.BlockSpec((1, D), lambda i: (0, 0)),
            pl.BlockSpec((R, D), lambda i: (i, 0)),
        ],
        out_specs=pl.BlockSpec((R, D), lambda i: (i, 0)),
        out_shape=jax.ShapeDtypeStruct((N, D), jnp.float32),
    )(agg, deg, b, s)


def kernel(x, edge_index, W1l, b1, W1r, W2l, b2, W2r, W3l, b3, W3r):
    pad = E_PAD - E
    pad_src = (jnp.arange(pad, dtype=jnp.int32) * 37) % N
    pad_dst = N + (jnp.arange(pad, dtype=jnp.int32) % (N_PAD - N))
    src = jnp.concatenate([edge_index[0], pad_src])
    dst = jnp.concatenate([edge_index[1], pad_dst])
    src_r = src.reshape(NW, CH, CHUNK)
    dst_r = dst.reshape(NW, CH, CHUNK)

    zrows = jnp.zeros((RPT, D), jnp.float32)
    zdeg = jnp.zeros((RPT, DEGW), jnp.float32)
    ones = jnp.ones((CHUNK, DEGW), jnp.float32)

    b1r = b1.reshape(1, D)
    b2r = b2.reshape(1, D)
    b3r = b3.reshape(1, D)

    sc_deg, sc_agg = _sc_kernels()
    dega = sc_deg(dst_r, zdeg, ones)
    z1, s1 = _tc_in(x, W1l, W1r)
    agg1 = sc_agg(z1, src_r, dst_r, zrows)
    z2, s2 = _tc_mid(agg1, dega, b1r, s1, W2l, W2r)
    agg2 = sc_agg(z2, src_r, dst_r, zrows)
    z3, s3 = _tc_mid(agg2, dega, b2r, s2, W3l, W3r)
    agg3 = sc_agg(z3, src_r, dst_r, zrows)
    return _tc_final(agg3, dega, b3r, s3)


# speedup vs baseline: 3.1397x; 1.0385x over previous
import functools

import jax
import jax.numpy as jnp
from jax import lax
from jax.experimental import pallas as pl
from jax.experimental.pallas import tpu as pltpu
from jax.experimental.pallas import tpu_sc as plsc

N = 10000
D = 128
E = 320000

NC = 2
NS = 16
NW = NC * NS
CHUNK = 64
CHH = 40
CH = 4 * CHH
NBUF = 4
PRE = 3
E_PAD = NW * CH * CHUNK
N_PAD = 10112
RPT = N_PAD // NS
DEGW = 128


def _sc_agg_body(z_hbm, src_hbm, dst_hbm, zrows_hbm, agg_out,
                 src_v, dst_v, rows_v, agg_sh, sem):
    cid = lax.axis_index("c")
    sid = lax.axis_index("s")
    wid = sid * NC + cid
    base = sid * RPT

    pltpu.sync_copy(zrows_hbm, agg_sh.at[pl.ds(base, RPT)])
    plsc.subcore_barrier()

    for h in (0, 1, 2, 3):
        pltpu.sync_copy(src_hbm.at[wid, pl.ds(h * CHH, CHH)], src_v)
        pltpu.sync_copy(dst_hbm.at[wid, pl.ds(h * CHH, CHH)], dst_v)

        def chunk_body(j, carry):
            @pl.when(j < CHH)
            def _():
                off = (j % NBUF) * CHUNK
                pltpu.async_copy(z_hbm.at[src_v.at[j]],
                                 rows_v.at[pl.ds(off, CHUNK)], sem.at[j % NBUF])

            @pl.when(j >= PRE)
            def _():
                jp = j - PRE
                off = (jp % NBUF) * CHUNK
                buf = rows_v.at[pl.ds(off, CHUNK)]
                pltpu.make_async_copy(z_hbm.at[src_v.at[jp]], buf,
                                      sem.at[jp % NBUF]).wait()
                pltpu.sync_copy(buf, agg_sh.at[dst_v.at[jp]], add=True)

            return carry

        lax.fori_loop(0, CHH + PRE, chunk_body, 0)

    plsc.subcore_barrier()
    pltpu.sync_copy(agg_sh.at[pl.ds(base, RPT)], agg_out.at[cid, pl.ds(base, RPT)])


def _sc_deg_body(dst_hbm, zdeg_hbm, ones_hbm, deg_out,
                 dst_v, ones_v, deg_sh):
    cid = lax.axis_index("c")
    sid = lax.axis_index("s")
    wid = sid * NC + cid
    base = sid * RPT

    pltpu.sync_copy(zdeg_hbm, deg_sh.at[pl.ds(base, RPT)])
    pltpu.sync_copy(ones_hbm, ones_v)
    pltpu.sync_copy(dst_hbm.at[wid], dst_v)
    plsc.subcore_barrier()

    def chunk_body(j, carry):
        pltpu.sync_copy(ones_v, deg_sh.at[dst_v.at[j]], add=True)
        return carry

    lax.fori_loop(0, CH, chunk_body, 0)
    plsc.subcore_barrier()
    pltpu.sync_copy(deg_sh.at[pl.ds(base, RPT)], deg_out.at[cid, pl.ds(base, RPT)])


@functools.lru_cache(maxsize=None)
def _sc_kernels():
    mesh = plsc.VectorSubcoreMesh(
        core_axis_name="c", subcore_axis_name="s", num_cores=NC, num_subcores=NS)
    sc_deg = pl.kernel(
        _sc_deg_body,
        out_type=jax.ShapeDtypeStruct((NC, N_PAD, DEGW), jnp.float32),
        mesh=mesh,
        scratch_types=[
            pltpu.VMEM((CH, CHUNK), jnp.int32),
            pltpu.VMEM((CHUNK, DEGW), jnp.float32),
            pltpu.VMEM_SHARED((N_PAD, DEGW), jnp.float32),
        ],
    )
    sc_agg = pl.kernel(
        _sc_agg_body,
        out_type=jax.ShapeDtypeStruct((NC, N_PAD, D), jnp.float32),
        mesh=mesh,
        scratch_types=[
            pltpu.VMEM((CHH, CHUNK), jnp.int32),
            pltpu.VMEM((CHH, CHUNK), jnp.int32),
            pltpu.VMEM((NBUF * CHUNK, D), jnp.float32),
            pltpu.VMEM_SHARED((N_PAD, D), jnp.float32),
            pltpu.SemaphoreType.DMA((NBUF,)),
        ],
    )
    return sc_deg, sc_agg


R = 2000


def _tc_in_body(x_ref, wl_ref, wr_ref, z_ref, s_ref):
    h = x_ref[...]
    z_ref[...] = jnp.dot(h, wl_ref[...], preferred_element_type=jnp.float32)
    s_ref[...] = jnp.dot(h, wr_ref[...], preferred_element_type=jnp.float32)


def _tc_in(x, wl, wr):
    return pl.pallas_call(
        _tc_in_body,
        grid=(N // R,),
        in_specs=[
            pl.BlockSpec((R, D), lambda i: (i, 0)),
            pl.BlockSpec((D, D), lambda i: (0, 0)),
            pl.BlockSpec((D, D), lambda i: (0, 0)),
        ],
        out_specs=[
            pl.BlockSpec((R, D), lambda i: (i, 0)),
            pl.BlockSpec((R, D), lambda i: (i, 0)),
        ],
        out_shape=[
            jax.ShapeDtypeStruct((N, D), jnp.float32),
            jax.ShapeDtypeStruct((N, D), jnp.float32),
        ],
    )(x, wl, wr)


def _combine_mean(agg_ref, deg_ref, b_ref, s_ref):
    a = agg_ref[...]
    dg = deg_ref[...]
    aggsum = a[0] + a[1]
    deg = dg[0, :, 0:1] + dg[1, :, 0:1]
    mean = aggsum / jnp.maximum(deg, 1.0)
    return mean + b_ref[...] + s_ref[...]


def _tc_mid_body(agg_ref, deg_ref, b_ref, s_ref, wl_ref, wr_ref, z_ref, s2_ref):
    h = jnp.maximum(_combine_mean(agg_ref, deg_ref, b_ref, s_ref), 0.0)
    z_ref[...] = jnp.dot(h, wl_ref[...], preferred_element_type=jnp.float32)
    s2_ref[...] = jnp.dot(h, wr_ref[...], preferred_element_type=jnp.float32)


def _tc_mid(agg, deg, b, s, wl, wr):
    return pl.pallas_call(
        _tc_mid_body,
        grid=(N // R,),
        in_specs=[
            pl.BlockSpec((NC, R, D), lambda i: (0, i, 0)),
            pl.BlockSpec((NC, R, DEGW), lambda i: (0, i, 0)),
            pl.BlockSpec((1, D), lambda i: (0, 0)),
            pl.BlockSpec((R, D), lambda i: (i, 0)),
            pl.BlockSpec((D, D), lambda i: (0, 0)),
            pl.BlockSpec((D, D), lambda i: (0, 0)),
        ],
        out_specs=[
            pl.BlockSpec((R, D), lambda i: (i, 0)),
            pl.BlockSpec((R, D), lambda i: (i, 0)),
        ],
        out_shape=[
            jax.ShapeDtypeStruct((N, D), jnp.float32),
            jax.ShapeDtypeStruct((N, D), jnp.float32),
        ],
    )(agg, deg, b, s, wl, wr)


def _tc_final_body(agg_ref, deg_ref, b_ref, s_ref, out_ref):
    pre = _combine_mean(agg_ref, deg_ref, b_ref, s_ref)
    m = jnp.max(pre, axis=-1, keepdims=True)
    e = jnp.exp(pre - m)
    out_ref[...] = e / jnp.sum(e, axis=-1, keepdims=True)


def _tc_final(agg, deg, b, s):
    return pl.pallas_call(
        _tc_final_body,
        grid=(N // R,),
        in_specs=[
            pl.BlockSpec((NC, R, D), lambda i: (0, i, 0)),
            pl.BlockSpec((NC, R, DEGW), lambda i: (0, i, 0)),
            pl.BlockSpec((1, D), lambda i: (0, 0)),
            pl.BlockSpec((R, D), lambda i: (i, 0)),
        ],
        out_specs=pl.BlockSpec((R, D), lambda i: (i, 0)),
        out_shape=jax.ShapeDtypeStruct((N, D), jnp.float32),
    )(agg, deg, b, s)


def kernel(x, edge_index, W1l, b1, W1r, W2l, b2, W2r, W3l, b3, W3r):
    pad = E_PAD - E
    pad_src = (jnp.arange(pad, dtype=jnp.int32) * 37) % N
    pad_dst = N + (jnp.arange(pad, dtype=jnp.int32) % (N_PAD - N))
    src = jnp.concatenate([edge_index[0], pad_src])
    dst = jnp.concatenate([edge_index[1], pad_dst])
    src_r = src.reshape(NW, CH, CHUNK)
    dst_r = dst.reshape(NW, CH, CHUNK)

    zrows = jnp.zeros((RPT, D), jnp.float32)
    zdeg = jnp.zeros((RPT, DEGW), jnp.float32)
    ones = jnp.ones((CHUNK, DEGW), jnp.float32)

    b1r = b1.reshape(1, D)
    b2r = b2.reshape(1, D)
    b3r = b3.reshape(1, D)

    sc_deg, sc_agg = _sc_kernels()
    dega = sc_deg(dst_r, zdeg, ones)
    z1, s1 = _tc_in(x, W1l, W1r)
    agg1 = sc_agg(z1, src_r, dst_r, zrows)
    z2, s2 = _tc_mid(agg1, dega, b1r, s1, W2l, W2r)
    agg2 = sc_agg(z2, src_r, dst_r, zrows)
    z3, s3 = _tc_mid(agg2, dega, b2r, s2, W3l, W3r)
    agg3 = sc_agg(z3, src_r, dst_r, zrows)
    return _tc_final(agg3, dega, b3r, s3)
